# Initial kernel scaffold; baseline (speedup 1.0000x reference)
import jax
import jax.numpy as jnp
from jax.experimental import pallas as pl


def kernel(x, edge_index, edge_attr, rel_type, weight, shared_W, attn_W):
    raise NotImplementedError("write your pallas kernel here")


# speedup vs baseline: 4.6800x; 4.6800x over previous
import functools

import jax
import jax.numpy as jnp
from jax import lax
from jax.experimental import pallas as pl
from jax.experimental.pallas import tpu as pltpu
from jax.experimental.pallas import tpu_sc as plsc

N = 10000
E = 160000
D = 128
R = 8

NC = 2
NS = 16
NW = NC * NS

CH = 128
NCH = 40
EPW = CH * NCH
EP = EPW * NW

NPAD = 10240
ROWS_PER_TILE = NPAD // NS
L = 16


def _uvw_body(a_ref, s_ref, o_ref):
    o_ref[...] = jnp.dot(a_ref[...], s_ref[...],
                         preferred_element_type=jnp.float32)


def _y_body(x_ref, w_ref, o_ref):
    o_ref[...] = jnp.dot(x_ref[...], w_ref[0],
                         preferred_element_type=jnp.float32)


def _s16_body(x_ref, u_ref, w_ref, su_ref, sw_ref):
    su_ref[...] = jnp.dot(x_ref[...], u_ref[...],
                          preferred_element_type=jnp.float32)
    sw_ref[...] = jnp.dot(x_ref[...], w_ref[...],
                          preferred_element_type=jnp.float32)


def _t16_body(ea_ref, v_ref, o_ref):
    o_ref[...] = jnp.dot(ea_ref[...], v_ref[...],
                         preferred_element_type=jnp.float32)


def _gidx_body(r_ref, s_ref, o_ref):
    o_ref[...] = r_ref[...] * N + s_ref[...]


def _add_body(p0_ref, p1_ref, o_ref):
    o_ref[...] = p0_ref[...] + p1_ref[...]


_mesh = plsc.VectorSubcoreMesh(core_axis_name="c", subcore_axis_name="s",
                               num_cores=NC, num_subcores=NS)


@functools.partial(
    pl.kernel,
    out_type=jax.ShapeDtypeStruct((NC * NPAD, D), jnp.float32),
    mesh=_mesh,
    compiler_params=pltpu.CompilerParams(use_tc_tiling_on_sc=False),
    scratch_types=[
        pltpu.VMEM((NCH, CH), jnp.int32),
        pltpu.VMEM((NCH, CH), jnp.int32),
        pltpu.VMEM((NCH, CH), jnp.int32),
        pltpu.VMEM((CH, L), jnp.float32),
        pltpu.VMEM((CH, D), jnp.float32),
        pltpu.VMEM((CH, L), jnp.float32),
        pltpu.VMEM((CH, L), jnp.float32),
        pltpu.VMEM_SHARED((NPAD, D), jnp.float32),
        pltpu.SemaphoreType.DMA,
        pltpu.SemaphoreType.DMA,
        pltpu.SemaphoreType.DMA,
    ],
)
def _sc_aggregate(y_hbm, su_hbm, sw_hbm, t_hbm, gidx_hbm, src_hbm, dst_hbm,
                  z_hbm, out_hbm,
                  gidx_v, src_v, dst_v, t_r, rows_v, su_r, sw_r, acc,
                  sem0, sem1, sem2):
    cid = lax.axis_index("c")
    sid = lax.axis_index("s")
    wid = sid * NC + cid

    pltpu.sync_copy(z_hbm, acc.at[pl.ds(sid * ROWS_PER_TILE, ROWS_PER_TILE)])

    pltpu.sync_copy(gidx_hbm.at[wid], gidx_v)
    pltpu.sync_copy(src_hbm.at[wid], src_v)
    pltpu.sync_copy(dst_hbm.at[wid], dst_v)
    plsc.subcore_barrier()

    def _chunk(c, carry):
        g0 = pltpu.async_copy(y_hbm.at[gidx_v.at[c]], rows_v, sem0)
        g1 = pltpu.async_copy(su_hbm.at[src_v.at[c]], su_r, sem1)
        g2 = pltpu.async_copy(sw_hbm.at[dst_v.at[c]], sw_r, sem2)
        pltpu.sync_copy(t_hbm.at[wid, c], t_r)
        g0.wait()
        g1.wait()
        g2.wait()

        def _edge(j, carry2):
            e16 = su_r[j, :] + sw_r[j, :] + t_r[j, :]
            for k in range(D // L):
                ks = pl.ds(k * L, L)
                rows_v[j, ks] = rows_v[j, ks] * e16
            return carry2

        lax.fori_loop(0, CH, _edge, 0)

        pltpu.sync_copy(rows_v, acc.at[dst_v.at[c]], add=True)
        return carry

    lax.fori_loop(0, NCH, _chunk, 0)
    plsc.subcore_barrier()

    row = sid * ROWS_PER_TILE
    pltpu.sync_copy(acc.at[pl.ds(row, ROWS_PER_TILE)],
                    out_hbm.at[pl.ds(cid * NPAD + row, ROWS_PER_TILE)])


def kernel(x, edge_index, edge_attr, rel_type, weight, shared_W, attn_W):
    f32 = jnp.float32
    src = edge_index[0]
    dst = edge_index[1]

    pad_e = EP - E
    src_p = jnp.concatenate([src, jnp.full((pad_e,), N, jnp.int32)])
    dst_p = jnp.concatenate([dst, jnp.full((pad_e,), N, jnp.int32)])
    rel_p = jnp.concatenate([rel_type, jnp.zeros((pad_e,), jnp.int32)])
    ea_p = jnp.concatenate([edge_attr, jnp.zeros((pad_e, D), f32)], axis=0)
    x_p = jnp.concatenate([x, jnp.zeros((NPAD - N, D), f32)], axis=0)

    attn_r = attn_W.reshape(3, D)

    uvw = pl.pallas_call(
        _uvw_body,
        out_shape=jax.ShapeDtypeStruct((3, D), f32),
    )(attn_r, shared_W)

    y = pl.pallas_call(
        _y_body,
        grid=(R,),
        in_specs=[pl.BlockSpec((N, D), lambda r: (0, 0)),
                  pl.BlockSpec((1, D, D), lambda r: (r, 0, 0))],
        out_specs=pl.BlockSpec((N, D), lambda r: (r, 0)),
        out_shape=jax.ShapeDtypeStruct((R * N, D), f32),
    )(x, weight)

    u16 = jnp.broadcast_to(uvw[0][:, None], (D, L))
    w16 = jnp.broadcast_to(uvw[2][:, None], (D, L))

    nb = NPAD // 1024
    su16, sw16 = pl.pallas_call(
        _s16_body,
        grid=(nb,),
        in_specs=[pl.BlockSpec((1024, D), lambda i: (i, 0)),
                  pl.BlockSpec((D, L), lambda i: (0, 0)),
                  pl.BlockSpec((D, L), lambda i: (0, 0))],
        out_specs=[pl.BlockSpec((1024, L), lambda i: (i, 0)),
                   pl.BlockSpec((1024, L), lambda i: (i, 0))],
        out_shape=[jax.ShapeDtypeStruct((NPAD, L), f32),
                   jax.ShapeDtypeStruct((NPAD, L), f32)],
    )(x_p, u16, w16)

    v16 = jnp.broadcast_to(uvw[1][:, None], (D, L))
    tb = EP // 10
    t16 = pl.pallas_call(
        _t16_body,
        grid=(10,),
        in_specs=[pl.BlockSpec((tb, D), lambda i: (i, 0)),
                  pl.BlockSpec((D, L), lambda i: (0, 0))],
        out_specs=pl.BlockSpec((tb, L), lambda i: (i, 0)),
        out_shape=jax.ShapeDtypeStruct((EP, L), f32),
    )(ea_p, v16)

    gidx_m = pl.pallas_call(
        _gidx_body,
        out_shape=jax.ShapeDtypeStruct((EP // D, D), jnp.int32),
    )(rel_p.reshape(EP // D, D), src_p.reshape(EP // D, D))

    t4 = t16.reshape(NW, NCH, CH, L)
    gidx3 = gidx_m.reshape(NW, NCH, CH)
    src3 = src_p.reshape(NW, NCH, CH)
    dst3 = dst_p.reshape(NW, NCH, CH)

    zeros_blk = jnp.zeros((ROWS_PER_TILE, D), f32)

    partials = _sc_aggregate(y, su16, sw16, t4, gidx3, src3, dst3,
                             zeros_blk)

    out = pl.pallas_call(
        _add_body,
        grid=(10,),
        in_specs=[pl.BlockSpec((1024, D), lambda i: (i, 0)),
                  pl.BlockSpec((1024, D), lambda i: (i + NPAD // 1024, 0))],
        out_specs=pl.BlockSpec((1024, D), lambda i: (i, 0)),
        out_shape=jax.ShapeDtypeStruct((N, D), f32),
    )(partials, partials)
    return out
